# Initial kernel scaffold; baseline (speedup 1.0000x reference)
import jax
import jax.numpy as jnp
from jax.experimental import pallas as pl


def kernel(mlmm_distances, mlmm_atomic_charges, mlmm_idxu, mlmm_idxv, mlmm_vectors, atomic_dipoles):
    raise NotImplementedError("write your pallas kernel here")


# speedup vs baseline: 15.0425x; 15.0425x over previous
import functools

import jax
import jax.numpy as jnp
from jax import lax
from jax.experimental import pallas as pl
from jax.experimental.pallas import tpu as pltpu
from jax.experimental.pallas import tpu_sc as plsc

_CUTOFF = 10.0
_CUTON = 2.5
_KE = 14.399645351950548

_N_NODES = 100000
_N_EDGES = 6400000

_NC = 2
_NS = 16
_NW = _NC * _NS
_E_PER_W = _N_EDGES // _NW
_CHUNK = 2000
_N_CHUNKS = _E_PER_W // _CHUNK
_L = 16


def _edge_kernel(dist_hbm, chg_hbm, idxu_hbm, idxv_hbm, vecf_hbm,
                 dxn_hbm, dyn_hbm, dzn_hbm,
                 out_hbm,
                 idxu_v, idxv_v, dist_v, vec_v, qi_v, qj_v,
                 dxi_v, dyi_v, dzi_v, out_v,
                 sem0, sem1, sem2, sem3, sem4):
    wid = lax.axis_index("s") * _NC + lax.axis_index("c")
    wbase = wid * _E_PER_W

    c2 = jnp.float32(_CUTOFF * _CUTOFF)
    on2 = jnp.float32(_CUTON * _CUTON)
    inv_den = jnp.float32(1.0 / (_CUTOFF**2 - _CUTON**2) ** 3)
    ke = jnp.float32(_KE)
    one = jnp.float32(1.0)
    zero = jnp.float32(0.0)
    cuton = jnp.float32(_CUTON)
    cutoff = jnp.float32(_CUTOFF)

    lane = lax.iota(jnp.int32, _L)

    def chunk_body(ci, _):
        base = wbase + ci * _CHUNK
        pltpu.sync_copy(idxu_hbm.at[pl.ds(base, _CHUNK)], idxu_v)
        pltpu.sync_copy(idxv_hbm.at[pl.ds(base, _CHUNK)], idxv_v)
        pltpu.sync_copy(dist_hbm.at[pl.ds(base, _CHUNK)], dist_v)
        pltpu.sync_copy(vecf_hbm.at[pl.ds(3 * base, 3 * _CHUNK)], vec_v)
        cp0 = pltpu.async_copy(chg_hbm.at[idxu_v], qi_v, sem0)
        cp1 = pltpu.async_copy(chg_hbm.at[idxv_v], qj_v, sem1)
        cp2 = pltpu.async_copy(dxn_hbm.at[idxu_v], dxi_v, sem2)
        cp3 = pltpu.async_copy(dyn_hbm.at[idxu_v], dyi_v, sem3)
        cp4 = pltpu.async_copy(dzn_hbm.at[idxu_v], dzi_v, sem4)
        cp0.wait()
        cp1.wait()
        cp2.wait()
        cp3.wait()
        cp4.wait()

        def vec_body(k, _):
            s = k * _L
            d = dist_v[pl.ds(s, _L)]
            qi = qi_v[pl.ds(s, _L)]
            qj = qj_v[pl.ds(s, _L)]
            dx = dxi_v[pl.ds(s, _L)]
            dy = dyi_v[pl.ds(s, _L)]
            dz = dzi_v[pl.ds(s, _L)]
            row3 = (s + lane) * 3
            vx = plsc.load_gather(vec_v, [row3])
            vy = plsc.load_gather(vec_v, [row3 + 1])
            vz = plsc.load_gather(vec_v, [row3 + 2])

            chi = one / d
            chi2 = chi * chi
            e = qi * qj * chi
            dot = vx * dx + vy * dy + vz * dz
            e = e + qj * dot * chi * chi2
            e = e * ke
            d2 = d * d
            t = c2 - d2
            sw = t * t * (c2 + jnp.float32(2.0) * d2 - jnp.float32(3.0) * on2) * inv_den
            sw = jnp.where(d < cuton, one, jnp.where(d > cutoff, zero, sw))
            out_v[pl.ds(s, _L)] = e * sw
            return 0

        lax.fori_loop(0, _CHUNK // _L, vec_body, 0, unroll=False)
        pltpu.sync_copy(out_v, out_hbm.at[pl.ds(base, _CHUNK)])
        return 0

    lax.fori_loop(0, _N_CHUNKS, chunk_body, 0, unroll=False)


@jax.jit
def _run(distances, charges, idxu, idxv, vectors, dipoles):
    vec_flat = vectors.reshape(-1)
    dxn = dipoles[:, 0]
    dyn = dipoles[:, 1]
    dzn = dipoles[:, 2]
    mesh = plsc.VectorSubcoreMesh(core_axis_name="c", subcore_axis_name="s")
    f = pl.kernel(
        _edge_kernel,
        out_type=jax.ShapeDtypeStruct((_N_EDGES,), jnp.float32),
        mesh=mesh,
        compiler_params=pltpu.CompilerParams(needs_layout_passes=False),
        scratch_types=[
            pltpu.VMEM((_CHUNK,), jnp.int32),
            pltpu.VMEM((_CHUNK,), jnp.int32),
            pltpu.VMEM((_CHUNK,), jnp.float32),
            pltpu.VMEM((3 * _CHUNK,), jnp.float32),
            pltpu.VMEM((_CHUNK,), jnp.float32),
            pltpu.VMEM((_CHUNK,), jnp.float32),
            pltpu.VMEM((_CHUNK,), jnp.float32),
            pltpu.VMEM((_CHUNK,), jnp.float32),
            pltpu.VMEM((_CHUNK,), jnp.float32),
            pltpu.VMEM((_CHUNK,), jnp.float32),
            pltpu.SemaphoreType.DMA,
            pltpu.SemaphoreType.DMA,
            pltpu.SemaphoreType.DMA,
            pltpu.SemaphoreType.DMA,
            pltpu.SemaphoreType.DMA,
        ],
    )
    return f(distances, charges, idxu, idxv, vec_flat, dxn, dyn, dzn)


def kernel(mlmm_distances, mlmm_atomic_charges, mlmm_idxu, mlmm_idxv,
           mlmm_vectors, atomic_dipoles):
    return _run(mlmm_distances, mlmm_atomic_charges, mlmm_idxu, mlmm_idxv,
                mlmm_vectors, atomic_dipoles)
